# baseline (device time: 49027 ns/iter reference)
import jax
import jax.numpy as jnp
from jax import lax
from jax.experimental import pallas as pl
from jax.experimental.pallas import tpu as pltpu

N_DEV = 4


def kernel(x, router_W, route_idx, expert_W, shared_W):
    n_tok, d_model = x.shape
    d_out = shared_W.shape[1]
    n_local = expert_W.shape[0]

    def body(x_ref, rw_ref, idx_ref, ew_ref, sw_ref, out_ref,
             comm_ref, send_sems, recv_sems):
        my = lax.axis_index("i")
        left = lax.rem(my + N_DEV - 1, N_DEV)
        right = lax.rem(my + 1, N_DEV)

        barrier_sem = pltpu.get_barrier_semaphore()
        for nbr in (left, right):
            pl.semaphore_signal(
                barrier_sem, inc=1,
                device_id=(nbr,), device_id_type=pl.DeviceIdType.MESH,
            )
        pl.semaphore_wait(barrier_sem, 2)

        xv = x_ref[:, :]

        scores = jnp.dot(xv, rw_ref[:, :], preferred_element_type=jnp.float32)
        m = jnp.max(scores, axis=-1, keepdims=True)
        p = jnp.exp(scores - m)
        probs = p / jnp.sum(p, axis=-1, keepdims=True)

        idx = idx_ref[:, :]
        eids = lax.broadcasted_iota(jnp.int32, scores.shape, 1)
        prob_sel = jnp.sum(
            jnp.where(eids == idx, probs, 0.0), axis=-1, keepdims=True
        )

        partial = jnp.zeros((n_tok, d_out), dtype=jnp.float32)
        for j in range(n_local):
            e = my * n_local + j
            coeff = jnp.where(idx == e, prob_sel, 0.0)
            partial = partial + jnp.dot(
                xv * coeff, ew_ref[j], preferred_element_type=jnp.float32
            )

        comm_ref[0, :, :] = partial

        rdma0 = pltpu.make_async_remote_copy(
            src_ref=comm_ref.at[0],
            dst_ref=comm_ref.at[1],
            send_sem=send_sems.at[0],
            recv_sem=recv_sems.at[0],
            device_id=(right,),
            device_id_type=pl.DeviceIdType.MESH,
        )
        rdma0.start()

        shared = jnp.dot(xv, sw_ref[:, :], preferred_element_type=jnp.float32)
        out_ref[:, :] = partial + shared

        rdma0.wait()
        out_ref[:, :] = out_ref[:, :] + comm_ref[1, :, :]

        for hop in range(1, N_DEV - 1):
            rdma = pltpu.make_async_remote_copy(
                src_ref=comm_ref.at[hop],
                dst_ref=comm_ref.at[hop + 1],
                send_sem=send_sems.at[hop],
                recv_sem=recv_sems.at[hop],
                device_id=(right,),
                device_id_type=pl.DeviceIdType.MESH,
            )
            rdma.start()
            rdma.wait()
            out_ref[:, :] = out_ref[:, :] + comm_ref[hop + 1, :, :]

    return pl.pallas_call(
        body,
        out_shape=jax.ShapeDtypeStruct((n_tok, d_out), jnp.float32),
        in_specs=[
            pl.BlockSpec(memory_space=pltpu.VMEM),
            pl.BlockSpec(memory_space=pltpu.VMEM),
            pl.BlockSpec(memory_space=pltpu.VMEM),
            pl.BlockSpec(memory_space=pltpu.VMEM),
            pl.BlockSpec(memory_space=pltpu.VMEM),
        ],
        out_specs=pl.BlockSpec(memory_space=pltpu.VMEM),
        scratch_shapes=[
            pltpu.VMEM((N_DEV, n_tok, d_out), jnp.float32),
            pltpu.SemaphoreType.DMA((N_DEV - 1,)),
            pltpu.SemaphoreType.DMA((N_DEV - 1,)),
        ],
        compiler_params=pltpu.CompilerParams(collective_id=0),
    )(x, router_W, route_idx, expert_W, shared_W)


# device time: 25275 ns/iter; 1.9397x vs baseline; 1.9397x over previous
import jax
import jax.numpy as jnp
from jax import lax
from jax.experimental import pallas as pl
from jax.experimental.pallas import tpu as pltpu

N_DEV = 4


def kernel(x, router_W, route_idx, expert_W, shared_W):
    n_tok, d_model = x.shape
    d_out = shared_W.shape[1]
    n_local = expert_W.shape[0]
    chunk = n_tok // N_DEV

    def body(x_ref, rw_ref, idx_ref, ew_ref, sw_ref, out_ref,
             pbuf, rs_buf, ag_buf, r_buf, send_sems, recv_sems):
        my = lax.axis_index("i")

        barrier_sem = pltpu.get_barrier_semaphore()
        for dq in range(1, N_DEV):
            pl.semaphore_signal(
                barrier_sem, inc=1,
                device_id=(lax.rem(my + dq, N_DEV),),
                device_id_type=pl.DeviceIdType.MESH,
            )
        pl.semaphore_wait(barrier_sem, N_DEV - 1)

        xv = x_ref[:, :]

        scores = jnp.dot(xv, rw_ref[:, :], preferred_element_type=jnp.float32)
        m = jnp.max(scores, axis=-1, keepdims=True)
        p = jnp.exp(scores - m)
        probs = p / jnp.sum(p, axis=-1, keepdims=True)

        idx = idx_ref[:, :]
        eids = lax.broadcasted_iota(jnp.int32, scores.shape, 1)
        prob_sel = jnp.sum(
            jnp.where(eids == idx, probs, 0.0), axis=-1, keepdims=True
        )

        partial = jnp.zeros((n_tok, d_out), dtype=jnp.float32)
        for j in range(n_local):
            e = my * n_local + j
            coeff = jnp.where(idx == e, prob_sel, 0.0)
            partial = partial + jnp.dot(
                xv * coeff, ew_ref[j], preferred_element_type=jnp.float32
            )
        pbuf[:, :] = partial

        rs_rdmas = []
        for dq in range(1, N_DEV):
            q = lax.rem(my + dq, N_DEV)
            slot = (N_DEV - 1) - dq
            rdma = pltpu.make_async_remote_copy(
                src_ref=pbuf.at[pl.ds(q * chunk, chunk)],
                dst_ref=rs_buf.at[slot],
                send_sem=send_sems.at[0, slot],
                recv_sem=recv_sems.at[0, slot],
                device_id=(q,),
                device_id_type=pl.DeviceIdType.MESH,
            )
            rdma.start()
            rs_rdmas.append(rdma)

        shared = jnp.dot(xv, sw_ref[:, :], preferred_element_type=jnp.float32)
        out_ref[:, :] = shared

        for rdma in rs_rdmas:
            rdma.wait_recv()

        r_buf[:, :] = (
            pbuf[pl.ds(my * chunk, chunk), :]
            + rs_buf[0, :, :] + rs_buf[1, :, :] + rs_buf[2, :, :]
        )

        ag_rdmas = []
        for dq in range(1, N_DEV):
            q = lax.rem(my + dq, N_DEV)
            slot = (N_DEV - 1) - dq
            rdma = pltpu.make_async_remote_copy(
                src_ref=r_buf,
                dst_ref=ag_buf.at[slot],
                send_sem=send_sems.at[1, slot],
                recv_sem=recv_sems.at[1, slot],
                device_id=(q,),
                device_id_type=pl.DeviceIdType.MESH,
            )
            rdma.start()
            ag_rdmas.append(rdma)

        out_ref[pl.ds(my * chunk, chunk), :] = (
            out_ref[pl.ds(my * chunk, chunk), :] + r_buf[:, :]
        )

        for rdma in ag_rdmas:
            rdma.wait_recv()
        for r in range(N_DEV - 1):
            s = lax.rem(my + r + 1, N_DEV)
            out_ref[pl.ds(s * chunk, chunk), :] = (
                out_ref[pl.ds(s * chunk, chunk), :] + ag_buf[r, :, :]
            )

        for rdma in rs_rdmas + ag_rdmas:
            rdma.wait_send()

    return pl.pallas_call(
        body,
        out_shape=jax.ShapeDtypeStruct((n_tok, d_out), jnp.float32),
        in_specs=[
            pl.BlockSpec(memory_space=pltpu.VMEM),
            pl.BlockSpec(memory_space=pltpu.VMEM),
            pl.BlockSpec(memory_space=pltpu.VMEM),
            pl.BlockSpec(memory_space=pltpu.VMEM),
            pl.BlockSpec(memory_space=pltpu.VMEM),
        ],
        out_specs=pl.BlockSpec(memory_space=pltpu.VMEM),
        scratch_shapes=[
            pltpu.VMEM((n_tok, d_out), jnp.float32),
            pltpu.VMEM((N_DEV - 1, chunk, d_out), jnp.float32),
            pltpu.VMEM((N_DEV - 1, chunk, d_out), jnp.float32),
            pltpu.VMEM((chunk, d_out), jnp.float32),
            pltpu.SemaphoreType.DMA((2, N_DEV - 1)),
            pltpu.SemaphoreType.DMA((2, N_DEV - 1)),
        ],
        compiler_params=pltpu.CompilerParams(collective_id=0),
    )(x, router_W, route_idx, expert_W, shared_W)


# device time: 19251 ns/iter; 2.5467x vs baseline; 1.3129x over previous
import jax
import jax.numpy as jnp
from jax import lax
from jax.experimental import pallas as pl
from jax.experimental.pallas import tpu as pltpu

N_DEV = 4


def kernel(x, router_W, route_idx, expert_W, shared_W):
    n_tok, d_model = x.shape
    d_out = shared_W.shape[1]
    n_local = expert_W.shape[0]
    chunk = n_tok // N_DEV

    def body(x_ref, rw_ref, idx_ref, ew_ref, sw_ref, out_ref,
             psel_buf, sbuf, rs_buf, ag_buf, r_buf, send_sems, recv_sems):
        my = lax.axis_index("i")

        barrier_sem = pltpu.get_barrier_semaphore()
        for dq in range(1, N_DEV):
            pl.semaphore_signal(
                barrier_sem, inc=1,
                device_id=(lax.rem(my + dq, N_DEV),),
                device_id_type=pl.DeviceIdType.MESH,
            )
        pl.semaphore_wait(barrier_sem, N_DEV - 1)

        xv = x_ref[:, :]

        scores = jnp.dot(xv, rw_ref[:, :], preferred_element_type=jnp.float32)
        m = jnp.max(scores, axis=-1, keepdims=True)
        p = jnp.exp(scores - m)
        probs = p / jnp.sum(p, axis=-1, keepdims=True)
        eids = lax.broadcasted_iota(jnp.int32, scores.shape, 1)
        psel_buf[:, :] = jnp.sum(
            jnp.where(eids == idx_ref[:, :], probs, 0.0),
            axis=-1, keepdims=True,
        )

        def chunk_partial(q):
            qs = q * chunk
            xq = x_ref[pl.ds(qs, chunk), :]
            iq = idx_ref[pl.ds(qs, chunk), :]
            pq = psel_buf[pl.ds(qs, chunk), :]
            acc = jnp.zeros((chunk, d_out), dtype=jnp.float32)
            for j in range(n_local):
                coeff = jnp.where(iq == my * n_local + j, pq, 0.0)
                acc = acc + jnp.dot(
                    xq * coeff, ew_ref[j], preferred_element_type=jnp.float32
                )
            return acc

        rs_rdmas = []
        for dq in (2, 1, 3):
            q = lax.rem(my + dq, N_DEV)
            slot = (N_DEV - 1) - dq
            sbuf[slot, :, :] = chunk_partial(q).astype(jnp.bfloat16)
            rdma = pltpu.make_async_remote_copy(
                src_ref=sbuf.at[slot],
                dst_ref=rs_buf.at[slot],
                send_sem=send_sems.at[0, slot],
                recv_sem=recv_sems.at[0, slot],
                device_id=(q,),
                device_id_type=pl.DeviceIdType.MESH,
            )
            rdma.start()
            rs_rdmas.append(rdma)

        own = chunk_partial(my)
        shared = jnp.dot(xv, sw_ref[:, :], preferred_element_type=jnp.float32)
        out_ref[:, :] = shared

        for rdma in rs_rdmas:
            rdma.wait_recv()

        r_f32 = (
            own
            + rs_buf[0, :, :].astype(jnp.float32)
            + rs_buf[1, :, :].astype(jnp.float32)
            + rs_buf[2, :, :].astype(jnp.float32)
        )
        r_buf[:, :] = r_f32.astype(jnp.bfloat16)

        ag_rdmas = []
        for dq in (2, 1, 3):
            q = lax.rem(my + dq, N_DEV)
            slot = (N_DEV - 1) - dq
            rdma = pltpu.make_async_remote_copy(
                src_ref=r_buf,
                dst_ref=ag_buf.at[slot],
                send_sem=send_sems.at[1, slot],
                recv_sem=recv_sems.at[1, slot],
                device_id=(q,),
                device_id_type=pl.DeviceIdType.MESH,
            )
            rdma.start()
            ag_rdmas.append(rdma)

        out_ref[pl.ds(my * chunk, chunk), :] = (
            out_ref[pl.ds(my * chunk, chunk), :] + r_f32
        )

        for rdma in ag_rdmas:
            rdma.wait_recv()
        for r in range(N_DEV - 1):
            s = lax.rem(my + r + 1, N_DEV)
            out_ref[pl.ds(s * chunk, chunk), :] = (
                out_ref[pl.ds(s * chunk, chunk), :]
                + ag_buf[r, :, :].astype(jnp.float32)
            )

        for rdma in rs_rdmas + ag_rdmas:
            rdma.wait_send()

    return pl.pallas_call(
        body,
        out_shape=jax.ShapeDtypeStruct((n_tok, d_out), jnp.float32),
        in_specs=[
            pl.BlockSpec(memory_space=pltpu.VMEM),
            pl.BlockSpec(memory_space=pltpu.VMEM),
            pl.BlockSpec(memory_space=pltpu.VMEM),
            pl.BlockSpec(memory_space=pltpu.VMEM),
            pl.BlockSpec(memory_space=pltpu.VMEM),
        ],
        out_specs=pl.BlockSpec(memory_space=pltpu.VMEM),
        scratch_shapes=[
            pltpu.VMEM((n_tok, 1), jnp.float32),
            pltpu.VMEM((N_DEV - 1, chunk, d_out), jnp.bfloat16),
            pltpu.VMEM((N_DEV - 1, chunk, d_out), jnp.bfloat16),
            pltpu.VMEM((N_DEV - 1, chunk, d_out), jnp.bfloat16),
            pltpu.VMEM((chunk, d_out), jnp.bfloat16),
            pltpu.SemaphoreType.DMA((2, N_DEV - 1)),
            pltpu.SemaphoreType.DMA((2, N_DEV - 1)),
        ],
        compiler_params=pltpu.CompilerParams(collective_id=0),
    )(x, router_W, route_idx, expert_W, shared_W)
